# Initial kernel scaffold; baseline (speedup 1.0000x reference)
import jax
import jax.numpy as jnp
from jax.experimental import pallas as pl


def kernel(x, Wqkv, bqkv, Wproj, bproj):
    raise NotImplementedError("write your pallas kernel here")


# speedup vs baseline: 1.5061x; 1.5061x over previous
import jax
import jax.numpy as jnp
from jax.experimental import pallas as pl

_N = 197
_C = 384
_H = 6
_DH = 64
_GS = 49
_G = 4
_TOPK = 2


def _attn_kernel(x_ref, wqkv_ref, bqkv_ref, wproj_ref, bproj_ref, o_ref):
    xb = x_ref[0]
    qkv = (
        jnp.dot(xb, wqkv_ref[...], preferred_element_type=jnp.float32)
        + bqkv_ref[...]
    )
    scale = _DH ** -0.5

    col = jax.lax.broadcasted_iota(jnp.int32, (_N, _N), 1)
    row = jax.lax.broadcasted_iota(jnp.int32, (_N, _N), 0)
    grpcol = jnp.clip((col - 1) // _GS, 0, _G - 1)

    outs = []
    for h in range(_H):
        q = qkv[:, h * _DH:(h + 1) * _DH]
        k = qkv[:, _C + h * _DH:_C + (h + 1) * _DH]
        v = qkv[:, 2 * _C + h * _DH:2 * _C + (h + 1) * _DH]

        scores = []
        for g in range(_G):
            kg = k[1 + g * _GS:1 + (g + 1) * _GS, :]
            gmax = jnp.max(kg, axis=0, keepdims=True)
            gmin = jnp.min(kg, axis=0, keepdims=True)
            ew = jnp.maximum(q * gmax, q * gmin)
            scores.append(jnp.sum(ew, axis=1, keepdims=True))

        sel = []
        for g in range(_G):
            rank = jnp.zeros((_N, 1), jnp.float32)
            for j in range(_G):
                if j == g:
                    continue
                cmp = (scores[j] > scores[g]).astype(jnp.float32)
                if j < g:
                    cmp = cmp + (scores[j] == scores[g]).astype(jnp.float32)
                rank = rank + cmp
            sel.append((rank < _TOPK).astype(jnp.float32))

        km = jnp.zeros((_N, _N), jnp.float32)
        for g in range(_G):
            km = km + sel[g] * (grpcol == g).astype(jnp.float32)
        km = jnp.where(col == 0, 1.0, km)
        km = jnp.where(row == 0, 1.0, km)

        logits = jax.lax.dot_general(
            q, k, (((1,), (1,)), ((), ())), preferred_element_type=jnp.float32
        )
        logits = logits * km * scale
        m = jnp.max(logits, axis=1, keepdims=True)
        e = jnp.exp(logits - m)
        s = jnp.sum(e, axis=1, keepdims=True)
        attn = e / s
        outs.append(jnp.dot(attn, v, preferred_element_type=jnp.float32))

    out = jnp.concatenate(outs, axis=1)
    o_ref[0] = (
        jnp.dot(out, wproj_ref[...], preferred_element_type=jnp.float32)
        + bproj_ref[...]
    )


def kernel(x, Wqkv, bqkv, Wproj, bproj):
    Bsz = x.shape[0]
    wqkv_t = Wqkv.T
    wproj_t = Wproj.T
    bqkv2 = bqkv.reshape(1, -1)
    bproj2 = bproj.reshape(1, -1)
    return pl.pallas_call(
        _attn_kernel,
        grid=(Bsz,),
        in_specs=[
            pl.BlockSpec((1, _N, _C), lambda b: (b, 0, 0)),
            pl.BlockSpec((_C, 3 * _C), lambda b: (0, 0)),
            pl.BlockSpec((1, 3 * _C), lambda b: (0, 0)),
            pl.BlockSpec((_C, _C), lambda b: (0, 0)),
            pl.BlockSpec((1, _C), lambda b: (0, 0)),
        ],
        out_specs=pl.BlockSpec((1, _N, _C), lambda b: (b, 0, 0)),
        out_shape=jax.ShapeDtypeStruct(x.shape, x.dtype),
    )(x, wqkv_t, bqkv2, wproj_t, bproj2)
